# Initial kernel scaffold; baseline (speedup 1.0000x reference)
import jax
import jax.numpy as jnp
from jax.experimental import pallas as pl


def kernel(x, edge_index, batch, W1, b1, W2, b2):
    raise NotImplementedError("write your pallas kernel here")


# speedup vs baseline: 69.0379x; 69.0379x over previous
import functools

import jax
import jax.numpy as jnp
from jax import lax
from jax.experimental import pallas as pl
from jax.experimental.pallas import tpu as pltpu
from jax.experimental.pallas import tpu_sc as plsc

N = 100000
E = 3200000
G = 128
H = 16
NC = 2
NS = 16
NW = NC * NS

K = 16
GR = 49
EPT = GR * K * 128
EPAD = NW * EPT
NPT = 6256
NPAD = NS * NPT

_mesh = plsc.VectorSubcoreMesh(core_axis_name="c", subcore_axis_name="s")


@functools.partial(
    pl.kernel,
    out_type=(jax.ShapeDtypeStruct((NPAD,), jnp.float32),
              jax.ShapeDtypeStruct((NPAD,), jnp.float32)),
    mesh=_mesh,
    scratch_types=[
        pltpu.VMEM((K, 128), jnp.int32),
        pltpu.VMEM((128,), jnp.float32),
        pltpu.VMEM((NPT,), jnp.float32),
        pltpu.VMEM_SHARED((NPAD,), jnp.float32),
    ],
)
def _sc_degree(dst_hbm, outa, outb, didx, ones, stage, acc):
    c = lax.axis_index("c")
    s = lax.axis_index("s")
    for i in range(8):
        ones[pl.ds(i * 16, 16)] = jnp.ones((16,), jnp.float32)

    def zfill(i, carry):
        stage[pl.ds(i * 16, 16)] = jnp.zeros((16,), jnp.float32)
        return carry

    lax.fori_loop(0, NPT // 16, zfill, 0)
    row0 = s * NPT
    pltpu.sync_copy(stage, acc.at[pl.ds(row0, NPT)])
    plsc.subcore_barrier()

    base = (c * NS + s) * (GR * K)

    def grp(g, carry):
        pltpu.sync_copy(dst_hbm.at[pl.ds(base + g * K, K)], didx)
        for j in range(K):
            pltpu.sync_copy(ones, acc.at[didx.at[j]], add=True)
        return carry

    lax.fori_loop(0, GR, grp, 0)
    plsc.subcore_barrier()

    pltpu.sync_copy(acc.at[pl.ds(row0, NPT)], stage)

    @pl.when(c == 0)
    def _():
        pltpu.sync_copy(stage, outa.at[pl.ds(row0, NPT)])

    @pl.when(c == 1)
    def _():
        pltpu.sync_copy(stage, outb.at[pl.ds(row0, NPT)])


@functools.partial(
    pl.kernel,
    out_type=(jax.ShapeDtypeStruct((NPAD, 8), jnp.float32),
              jax.ShapeDtypeStruct((NPAD, 8), jnp.float32)),
    mesh=_mesh,
    scratch_types=[
        pltpu.VMEM((K, 128), jnp.int32),
        pltpu.VMEM((K, 128), jnp.int32),
        pltpu.VMEM((K, 128, 8), jnp.float32),
        pltpu.VMEM((NPT, 8), jnp.float32),
        pltpu.VMEM_SHARED((NPAD, 8), jnp.float32),
        pltpu.SemaphoreType.DMA,
    ],
    compiler_params=pltpu.CompilerParams(use_tc_tiling_on_sc=False),
)
def _sc_agg8(src_hbm, dst_hbm, u_hbm, z8_hbm, outa, outb,
             sidx, didx, rows, stage, acc, gsem):
    c = lax.axis_index("c")
    s = lax.axis_index("s")
    row0 = s * NPT
    pltpu.sync_copy(z8_hbm, stage)
    pltpu.sync_copy(stage, acc.at[pl.ds(row0, NPT), :])
    plsc.subcore_barrier()

    base = (c * NS + s) * (GR * K)

    def grp(g, carry):
        pltpu.sync_copy(src_hbm.at[pl.ds(base + g * K, K)], sidx)
        pltpu.sync_copy(dst_hbm.at[pl.ds(base + g * K, K)], didx)
        handles = [pltpu.async_copy(u_hbm.at[sidx.at[j]], rows.at[j], gsem)
                   for j in range(K)]
        for h in handles:
            h.wait()
        for j in range(K):
            pltpu.sync_copy(rows.at[j], acc.at[didx.at[j]], add=True)
        return carry

    lax.fori_loop(0, GR, grp, 0)
    plsc.subcore_barrier()

    pltpu.sync_copy(acc.at[pl.ds(row0, NPT), :], stage)

    @pl.when(c == 0)
    def _():
        pltpu.sync_copy(stage, outa.at[pl.ds(row0, NPT), :])

    @pl.when(c == 1)
    def _():
        pltpu.sync_copy(stage, outb.at[pl.ds(row0, NPT), :])


@functools.partial(
    pl.kernel,
    out_type=(jax.ShapeDtypeStruct((NPAD,), jnp.float32),
              jax.ShapeDtypeStruct((NPAD,), jnp.float32)),
    mesh=_mesh,
    scratch_types=[
        pltpu.VMEM((K, 128), jnp.int32),
        pltpu.VMEM((K, 128), jnp.int32),
        pltpu.VMEM((K, 128), jnp.float32),
        pltpu.VMEM((NPT,), jnp.float32),
        pltpu.VMEM_SHARED((NPAD,), jnp.float32),
        pltpu.SemaphoreType.DMA,
    ],
)
def _sc_agg1(src_hbm, dst_hbm, g2_hbm, outa, outb,
             sidx, didx, rows, stage, acc, gsem):
    c = lax.axis_index("c")
    s = lax.axis_index("s")

    def zfill(i, carry):
        stage[pl.ds(i * 16, 16)] = jnp.zeros((16,), jnp.float32)
        return carry

    lax.fori_loop(0, NPT // 16, zfill, 0)
    row0 = s * NPT
    pltpu.sync_copy(stage, acc.at[pl.ds(row0, NPT)])
    plsc.subcore_barrier()

    base = (c * NS + s) * (GR * K)

    def grp(g, carry):
        pltpu.sync_copy(src_hbm.at[pl.ds(base + g * K, K)], sidx)
        pltpu.sync_copy(dst_hbm.at[pl.ds(base + g * K, K)], didx)
        handles = [pltpu.async_copy(g2_hbm.at[sidx.at[j]], rows.at[j], gsem)
                   for j in range(K)]
        for h in handles:
            h.wait()
        for j in range(K):
            pltpu.sync_copy(rows.at[j], acc.at[didx.at[j]], add=True)
        return carry

    lax.fori_loop(0, GR, grp, 0)
    plsc.subcore_barrier()

    pltpu.sync_copy(acc.at[pl.ds(row0, NPT)], stage)

    @pl.when(c == 0)
    def _():
        pltpu.sync_copy(stage, outa.at[pl.ds(row0, NPT)])

    @pl.when(c == 1)
    def _():
        pltpu.sync_copy(stage, outb.at[pl.ds(row0, NPT)])


def _tc1_body(dega_ref, degb_ref, x_ref, dinv_ref, u_ref):
    deg = dega_ref[:, :] + degb_ref[:, :] + 1.0
    dinv = lax.rsqrt(deg)
    dinv_ref[:, :] = dinv
    u_ref[:, :] = dinv * x_ref[:, :]


def _tc1(dega, degb, xp):
    return pl.pallas_call(
        _tc1_body,
        grid=(NS,),
        in_specs=[
            pl.BlockSpec((NPT, 1), lambda i: (i, 0)),
            pl.BlockSpec((NPT, 1), lambda i: (i, 0)),
            pl.BlockSpec((NPT, 8), lambda i: (i, 0)),
        ],
        out_specs=[
            pl.BlockSpec((NPT, 1), lambda i: (i, 0)),
            pl.BlockSpec((NPT, 8), lambda i: (i, 0)),
        ],
        out_shape=[
            jax.ShapeDtypeStruct((NPAD, 1), jnp.float32),
            jax.ShapeDtypeStruct((NPAD, 8), jnp.float32),
        ],
    )(dega, degb, xp)


def _tc2_body(dinv_ref, u_ref, s1a_ref, s1b_ref, w1_ref, b1_ref, w2_ref,
              g2_ref):
    dinv = dinv_ref[:, :]
    t = dinv * (s1a_ref[:, :] + s1b_ref[:, :] + u_ref[:, :])
    h = jnp.dot(t, w1_ref[:, :], preferred_element_type=jnp.float32)
    h = jnp.maximum(h + b1_ref[:, :], 0.0)
    g2 = jnp.dot(h, w2_ref[:, :], preferred_element_type=jnp.float32)
    g2_ref[:, :] = g2 * dinv


def _tc2(dinv, u, s1a, s1b, w1p, b1, w2):
    return pl.pallas_call(
        _tc2_body,
        grid=(NS,),
        in_specs=[
            pl.BlockSpec((NPT, 1), lambda i: (i, 0)),
            pl.BlockSpec((NPT, 8), lambda i: (i, 0)),
            pl.BlockSpec((NPT, 8), lambda i: (i, 0)),
            pl.BlockSpec((NPT, 8), lambda i: (i, 0)),
            pl.BlockSpec((8, H), lambda i: (0, 0)),
            pl.BlockSpec((1, H), lambda i: (0, 0)),
            pl.BlockSpec((H, 1), lambda i: (0, 0)),
        ],
        out_specs=pl.BlockSpec((NPT, 1), lambda i: (i, 0)),
        out_shape=jax.ShapeDtypeStruct((NPAD, 1), jnp.float32),
    )(dinv, u, s1a, s1b, w1p, b1, w2)


def _tc3_body(dinv_ref, g2_ref, s2a_ref, s2b_ref, batch_ref, b2_ref,
              out_ref, sums, counts):
    i = pl.program_id(0)
    o = dinv_ref[:, :] * (s2a_ref[:, :] + s2b_ref[:, :] + g2_ref[:, :]) \
        + b2_ref[0, 0]
    gid = lax.broadcasted_iota(jnp.int32, (o.shape[0], G), 1)
    onehot = (batch_ref[:, :] == gid).astype(jnp.float32)
    s_part = jnp.sum(o * onehot, axis=0)[None, :]
    c_part = jnp.sum(onehot, axis=0)[None, :]

    @pl.when(i == 0)
    def _():
        sums[:, :] = jnp.zeros((1, G), jnp.float32)
        counts[:, :] = jnp.zeros((1, G), jnp.float32)

    sums[:, :] += s_part
    counts[:, :] += c_part

    @pl.when(i == pl.num_programs(0) - 1)
    def _():
        pooled = sums[:, :] / jnp.maximum(counts[:, :], 1.0)
        out_ref[:, :] = jax.nn.sigmoid(pooled)


def _tc3(dinv, g2, s2a, s2b, batch2d, b2):
    br = 2000
    return pl.pallas_call(
        _tc3_body,
        grid=(N // br,),
        in_specs=[
            pl.BlockSpec((br, 1), lambda i: (i, 0)),
            pl.BlockSpec((br, 1), lambda i: (i, 0)),
            pl.BlockSpec((br, 1), lambda i: (i, 0)),
            pl.BlockSpec((br, 1), lambda i: (i, 0)),
            pl.BlockSpec((br, 1), lambda i: (i, 0)),
            pl.BlockSpec((1, 1), lambda i: (0, 0)),
        ],
        out_specs=pl.BlockSpec((1, G), lambda i: (0, 0)),
        out_shape=jax.ShapeDtypeStruct((1, G), jnp.float32),
        scratch_shapes=[
            pltpu.VMEM((1, G), jnp.float32),
            pltpu.VMEM((1, G), jnp.float32),
        ],
    )(dinv, g2, s2a, s2b, batch2d, b2)


def kernel(x, edge_index, batch, W1, b1, W2, b2):
    src = edge_index[0].astype(jnp.int32)
    dst = edge_index[1].astype(jnp.int32)
    pad = EPAD - E
    src2d = jnp.concatenate(
        [src, jnp.zeros((pad,), jnp.int32)]).reshape(EPAD // 128, 128)
    dst2d = jnp.concatenate(
        [dst, jnp.full((pad,), N, jnp.int32)]).reshape(EPAD // 128, 128)

    xp = jnp.zeros((NPAD, 8), jnp.float32).at[:N, :7].set(x)
    w1p = jnp.zeros((8, H), jnp.float32).at[:7, :].set(W1)
    b1r = b1.reshape(1, H)
    w2r = W2.reshape(H, 1)
    b2r = b2.reshape(1, 1)
    batch2d = batch.astype(jnp.int32).reshape(N, 1)

    z8 = jnp.zeros((NPT, 8), jnp.float32)
    dega, degb = _sc_degree(dst2d)
    dinv, u = _tc1(dega.reshape(NPAD, 1), degb.reshape(NPAD, 1), xp)
    s1a, s1b = _sc_agg8(src2d, dst2d, u, z8)
    g2 = _tc2(dinv, u, s1a, s1b, w1p, b1r, w2r)
    s2a, s2b = _sc_agg1(src2d, dst2d, g2.reshape(NPAD))
    out = _tc3(dinv, g2, s2a.reshape(NPAD, 1), s2b.reshape(NPAD, 1),
               batch2d, b2r)
    return out.reshape(G, 1)


# speedup vs baseline: 82.0854x; 1.1890x over previous
import functools

import jax
import jax.numpy as jnp
from jax import lax
from jax.experimental import pallas as pl
from jax.experimental.pallas import tpu as pltpu
from jax.experimental.pallas import tpu_sc as plsc

N = 100000
E = 3200000
G = 128
H = 16
NC = 2
NS = 16
NW = NC * NS

K = 8
GR = 98
EPT = GR * K * 128
EPAD = NW * EPT
NPT = 6256
NPAD = NS * NPT

_mesh = plsc.VectorSubcoreMesh(core_axis_name="c", subcore_axis_name="s")


@functools.partial(
    pl.kernel,
    out_type=(jax.ShapeDtypeStruct((NPAD,), jnp.float32),
              jax.ShapeDtypeStruct((NPAD,), jnp.float32)),
    mesh=_mesh,
    scratch_types=[
        pltpu.VMEM((2, K, 128), jnp.int32),
        pltpu.VMEM((128,), jnp.float32),
        pltpu.VMEM((NPT,), jnp.float32),
        pltpu.VMEM_SHARED((NPAD,), jnp.float32),
        pltpu.SemaphoreType.DMA,
    ],
    compiler_params=pltpu.CompilerParams(needs_layout_passes=False),
)
def _sc_degree(dst_hbm, outa, outb, didx, ones, stage, acc, ssem):
    c = lax.axis_index("c")
    s = lax.axis_index("s")
    for i in range(8):
        ones[pl.ds(i * 16, 16)] = jnp.ones((16,), jnp.float32)

    def zfill(i, carry):
        stage[pl.ds(i * 16, 16)] = jnp.zeros((16,), jnp.float32)
        return carry

    lax.fori_loop(0, NPT // 16, zfill, 0)
    row0 = s * NPT
    pltpu.sync_copy(stage, acc.at[pl.ds(row0, NPT)])
    plsc.subcore_barrier()

    base = (c * NS + s) * (GR * K)

    def pair(i, carry):
        g = i * 2
        pltpu.sync_copy(dst_hbm.at[pl.ds(base + g * K, K)], didx.at[0])
        sa = [pltpu.async_copy(ones, acc.at[didx.at[0, j]], ssem, add=True)
              for j in range(K)]
        pltpu.sync_copy(dst_hbm.at[pl.ds(base + (g + 1) * K, K)], didx.at[1])
        sb = [pltpu.async_copy(ones, acc.at[didx.at[1, j]], ssem, add=True)
              for j in range(K)]
        for h in sa + sb:
            h.wait()
        return carry

    lax.fori_loop(0, GR // 2, pair, 0)
    plsc.subcore_barrier()

    pltpu.sync_copy(acc.at[pl.ds(row0, NPT)], stage)

    @pl.when(c == 0)
    def _():
        pltpu.sync_copy(stage, outa.at[pl.ds(row0, NPT)])

    @pl.when(c == 1)
    def _():
        pltpu.sync_copy(stage, outb.at[pl.ds(row0, NPT)])


@functools.partial(
    pl.kernel,
    out_type=(jax.ShapeDtypeStruct((NPAD, 8), jnp.float32),
              jax.ShapeDtypeStruct((NPAD, 8), jnp.float32)),
    mesh=_mesh,
    scratch_types=[
        pltpu.VMEM((2, K, 128), jnp.int32),
        pltpu.VMEM((2, K, 128), jnp.int32),
        pltpu.VMEM((2, K, 128, 8), jnp.float32),
        pltpu.VMEM((NPT, 8), jnp.float32),
        pltpu.VMEM_SHARED((NPAD, 8), jnp.float32),
        pltpu.SemaphoreType.DMA,
        pltpu.SemaphoreType.DMA,
    ],
    compiler_params=pltpu.CompilerParams(use_tc_tiling_on_sc=False,
                                         needs_layout_passes=False),
)
def _sc_agg8(src_hbm, dst_hbm, u_hbm, z8_hbm, outa, outb,
             sidx, didx, rows, stage, acc, gsem, ssem):
    c = lax.axis_index("c")
    s = lax.axis_index("s")
    row0 = s * NPT
    pltpu.sync_copy(z8_hbm, stage)
    pltpu.sync_copy(stage, acc.at[pl.ds(row0, NPT), :])
    plsc.subcore_barrier()

    base = (c * NS + s) * (GR * K)

    def load_idx(g, slot):
        pltpu.sync_copy(src_hbm.at[pl.ds(base + g * K, K)], sidx.at[slot])
        pltpu.sync_copy(dst_hbm.at[pl.ds(base + g * K, K)], didx.at[slot])

    def fire_gathers(slot):
        return [pltpu.async_copy(u_hbm.at[sidx.at[slot, j]],
                                 rows.at[slot, j], gsem)
                for j in range(K)]

    def fire_scatters(slot):
        return [pltpu.async_copy(rows.at[slot, j],
                                 acc.at[didx.at[slot, j]], ssem, add=True)
                for j in range(K)]

    def pair(i, carry):
        g = i * 2
        load_idx(g, 0)
        ga = fire_gathers(0)
        load_idx(g + 1, 1)
        for h in ga:
            h.wait()
        sa = fire_scatters(0)
        gb = fire_gathers(1)
        for h in gb:
            h.wait()
        sb = fire_scatters(1)
        for h in sa + sb:
            h.wait()
        return carry

    lax.fori_loop(0, GR // 2, pair, 0)
    plsc.subcore_barrier()

    pltpu.sync_copy(acc.at[pl.ds(row0, NPT), :], stage)

    @pl.when(c == 0)
    def _():
        pltpu.sync_copy(stage, outa.at[pl.ds(row0, NPT), :])

    @pl.when(c == 1)
    def _():
        pltpu.sync_copy(stage, outb.at[pl.ds(row0, NPT), :])


@functools.partial(
    pl.kernel,
    out_type=(jax.ShapeDtypeStruct((NPAD,), jnp.float32),
              jax.ShapeDtypeStruct((NPAD,), jnp.float32)),
    mesh=_mesh,
    scratch_types=[
        pltpu.VMEM((2, K, 128), jnp.int32),
        pltpu.VMEM((2, K, 128), jnp.int32),
        pltpu.VMEM((2, K, 128), jnp.float32),
        pltpu.VMEM((NPAD,), jnp.float32),
        pltpu.VMEM((NPT,), jnp.float32),
        pltpu.VMEM_SHARED((NPAD,), jnp.float32),
        pltpu.SemaphoreType.DMA,
    ],
    compiler_params=pltpu.CompilerParams(needs_layout_passes=False),
)
def _sc_agg1(src_hbm, dst_hbm, g2_hbm, outa, outb,
             sidx, didx, msgs, table, stage, acc, ssem):
    c = lax.axis_index("c")
    s = lax.axis_index("s")

    def zfill(i, carry):
        stage[pl.ds(i * 16, 16)] = jnp.zeros((16,), jnp.float32)
        return carry

    lax.fori_loop(0, NPT // 16, zfill, 0)
    row0 = s * NPT
    pltpu.sync_copy(stage, acc.at[pl.ds(row0, NPT)])
    pltpu.sync_copy(g2_hbm, table)
    plsc.subcore_barrier()

    base = (c * NS + s) * (GR * K)

    def gather_msgs(slot):
        for j in range(K):
            for jj in range(8):
                idx16 = sidx[slot, j, pl.ds(jj * 16, 16)]
                vals = plsc.load_gather(table, [idx16])
                msgs[slot, j, pl.ds(jj * 16, 16)] = vals

    def pair(i, carry):
        g = i * 2
        pltpu.sync_copy(src_hbm.at[pl.ds(base + g * K, K)], sidx.at[0])
        pltpu.sync_copy(dst_hbm.at[pl.ds(base + g * K, K)], didx.at[0])
        gather_msgs(0)
        sa = [pltpu.async_copy(msgs.at[0, j], acc.at[didx.at[0, j]], ssem,
                               add=True) for j in range(K)]
        pltpu.sync_copy(src_hbm.at[pl.ds(base + (g + 1) * K, K)], sidx.at[1])
        pltpu.sync_copy(dst_hbm.at[pl.ds(base + (g + 1) * K, K)], didx.at[1])
        gather_msgs(1)
        sb = [pltpu.async_copy(msgs.at[1, j], acc.at[didx.at[1, j]], ssem,
                               add=True) for j in range(K)]
        for h in sa + sb:
            h.wait()
        return carry

    lax.fori_loop(0, GR // 2, pair, 0)
    plsc.subcore_barrier()

    pltpu.sync_copy(acc.at[pl.ds(row0, NPT)], stage)

    @pl.when(c == 0)
    def _():
        pltpu.sync_copy(stage, outa.at[pl.ds(row0, NPT)])

    @pl.when(c == 1)
    def _():
        pltpu.sync_copy(stage, outb.at[pl.ds(row0, NPT)])


def _tc1_body(dega_ref, degb_ref, x_ref, dinv_ref, u_ref):
    deg = dega_ref[:, :] + degb_ref[:, :] + 1.0
    dinv = lax.rsqrt(deg)
    dinv_ref[:, :] = dinv
    u_ref[:, :] = dinv * x_ref[:, :]


def _tc1(dega, degb, xp):
    return pl.pallas_call(
        _tc1_body,
        grid=(NS,),
        in_specs=[
            pl.BlockSpec((NPT, 1), lambda i: (i, 0)),
            pl.BlockSpec((NPT, 1), lambda i: (i, 0)),
            pl.BlockSpec((NPT, 8), lambda i: (i, 0)),
        ],
        out_specs=[
            pl.BlockSpec((NPT, 1), lambda i: (i, 0)),
            pl.BlockSpec((NPT, 8), lambda i: (i, 0)),
        ],
        out_shape=[
            jax.ShapeDtypeStruct((NPAD, 1), jnp.float32),
            jax.ShapeDtypeStruct((NPAD, 8), jnp.float32),
        ],
    )(dega, degb, xp)


def _tc2_body(dinv_ref, u_ref, s1a_ref, s1b_ref, w1_ref, b1_ref, w2_ref,
              g2_ref):
    dinv = dinv_ref[:, :]
    t = dinv * (s1a_ref[:, :] + s1b_ref[:, :] + u_ref[:, :])
    h = jnp.dot(t, w1_ref[:, :], preferred_element_type=jnp.float32)
    h = jnp.maximum(h + b1_ref[:, :], 0.0)
    g2 = jnp.dot(h, w2_ref[:, :], preferred_element_type=jnp.float32)
    g2_ref[:, :] = g2 * dinv


def _tc2(dinv, u, s1a, s1b, w1p, b1, w2):
    return pl.pallas_call(
        _tc2_body,
        grid=(NS,),
        in_specs=[
            pl.BlockSpec((NPT, 1), lambda i: (i, 0)),
            pl.BlockSpec((NPT, 8), lambda i: (i, 0)),
            pl.BlockSpec((NPT, 8), lambda i: (i, 0)),
            pl.BlockSpec((NPT, 8), lambda i: (i, 0)),
            pl.BlockSpec((8, H), lambda i: (0, 0)),
            pl.BlockSpec((1, H), lambda i: (0, 0)),
            pl.BlockSpec((H, 1), lambda i: (0, 0)),
        ],
        out_specs=pl.BlockSpec((NPT, 1), lambda i: (i, 0)),
        out_shape=jax.ShapeDtypeStruct((NPAD, 1), jnp.float32),
    )(dinv, u, s1a, s1b, w1p, b1, w2)


def _tc3_body(dinv_ref, g2_ref, s2a_ref, s2b_ref, batch_ref, b2_ref,
              out_ref, sums, counts):
    i = pl.program_id(0)
    o = dinv_ref[:, :] * (s2a_ref[:, :] + s2b_ref[:, :] + g2_ref[:, :]) \
        + b2_ref[0, 0]
    gid = lax.broadcasted_iota(jnp.int32, (o.shape[0], G), 1)
    onehot = (batch_ref[:, :] == gid).astype(jnp.float32)
    s_part = jnp.sum(o * onehot, axis=0)[None, :]
    c_part = jnp.sum(onehot, axis=0)[None, :]

    @pl.when(i == 0)
    def _():
        sums[:, :] = jnp.zeros((1, G), jnp.float32)
        counts[:, :] = jnp.zeros((1, G), jnp.float32)

    sums[:, :] += s_part
    counts[:, :] += c_part

    @pl.when(i == pl.num_programs(0) - 1)
    def _():
        pooled = sums[:, :] / jnp.maximum(counts[:, :], 1.0)
        out_ref[:, :] = jax.nn.sigmoid(pooled)


def _tc3(dinv, g2, s2a, s2b, batch2d, b2):
    br = 10000
    return pl.pallas_call(
        _tc3_body,
        grid=(N // br,),
        in_specs=[
            pl.BlockSpec((br, 1), lambda i: (i, 0)),
            pl.BlockSpec((br, 1), lambda i: (i, 0)),
            pl.BlockSpec((br, 1), lambda i: (i, 0)),
            pl.BlockSpec((br, 1), lambda i: (i, 0)),
            pl.BlockSpec((br, 1), lambda i: (i, 0)),
            pl.BlockSpec((1, 1), lambda i: (0, 0)),
        ],
        out_specs=pl.BlockSpec((1, G), lambda i: (0, 0)),
        out_shape=jax.ShapeDtypeStruct((1, G), jnp.float32),
        scratch_shapes=[
            pltpu.VMEM((1, G), jnp.float32),
            pltpu.VMEM((1, G), jnp.float32),
        ],
    )(dinv, g2, s2a, s2b, batch2d, b2)


def kernel(x, edge_index, batch, W1, b1, W2, b2):
    src = edge_index[0].astype(jnp.int32)
    dst = edge_index[1].astype(jnp.int32)
    pad = EPAD - E
    src2d = jnp.concatenate(
        [src, jnp.zeros((pad,), jnp.int32)]).reshape(EPAD // 128, 128)
    dst2d = jnp.concatenate(
        [dst, jnp.full((pad,), N, jnp.int32)]).reshape(EPAD // 128, 128)

    xp = jnp.zeros((NPAD, 8), jnp.float32).at[:N, :7].set(x)
    w1p = jnp.zeros((8, H), jnp.float32).at[:7, :].set(W1)
    b1r = b1.reshape(1, H)
    w2r = W2.reshape(H, 1)
    b2r = b2.reshape(1, 1)
    batch2d = batch.astype(jnp.int32).reshape(N, 1)

    z8 = jnp.zeros((NPT, 8), jnp.float32)
    dega, degb = _sc_degree(dst2d)
    dinv, u = _tc1(dega.reshape(NPAD, 1), degb.reshape(NPAD, 1), xp)
    s1a, s1b = _sc_agg8(src2d, dst2d, u, z8)
    g2 = _tc2(dinv, u, s1a, s1b, w1p, b1r, w2r)
    s2a, s2b = _sc_agg1(src2d, dst2d, g2.reshape(NPAD))
    out = _tc3(dinv, g2, s2a.reshape(NPAD, 1), s2b.reshape(NPAD, 1),
               batch2d, b2r)
    return out.reshape(G, 1)
